# baseline (speedup 1.0000x reference)
import functools

import jax
import jax.numpy as jnp
from jax import lax
from jax.experimental import pallas as pl
from jax.experimental.pallas import tpu as pltpu
from jax.experimental.pallas import tpu_sc as plsc

N_NODES = 10000
D = 128
NPAD = 10240
N_REL = 6
E_TOTAL = 4 * 320000 + 2 * 40000
NW = 32
CHUNK = 128
GRP = 8
NBUF = 2
GLAG = 1 if NBUF < 4 else 2
CPTA = 264
CPTB = 404
CPT = (CPTA + CPTB) // 2
E_PAD = NW * CPT * CHUNK
NJUNK = NPAD - N_NODES
ROWS_PER_TILE = NPAD // 16
BLK = 512
NBLK = NPAD // BLK
EPS = 1e-5


def _gn_block(x, g, b):
    m = jnp.mean(x, axis=1, keepdims=True)
    v = jnp.mean((x - m) ** 2, axis=1, keepdims=True)
    return (x - m) * lax.rsqrt(v + EPS) * g + b


def _input_body(ctr_ref, ft_ref, w1c, b1c, w2c, gc, bc, w1s, b1s, w2s, gs, bs,
                out_ref):
    h = jnp.maximum(
        jnp.dot(ctr_ref[...], w1c[...], preferred_element_type=jnp.float32,
                precision=lax.Precision.HIGHEST) + b1c[...], 0.0)
    h = _gn_block(
        jnp.dot(h, w2c[...], preferred_element_type=jnp.float32,
                precision=lax.Precision.HIGHEST), gc[...], bc[...])
    s = jnp.maximum(
        jnp.dot(ft_ref[...], w1s[...], preferred_element_type=jnp.float32,
                precision=lax.Precision.HIGHEST) + b1s[...], 0.0)
    s = _gn_block(
        jnp.dot(s, w2s[...], preferred_element_type=jnp.float32,
                precision=lax.Precision.HIGHEST), gs[...], bs[...])
    out_ref[...] = jnp.maximum(h + s, 0.0)


def _input_stage(ctrs_p, feats_p, w1c, b1c, w2c, gc, bc, w1s, b1s, w2s, gs, bs):
    full = lambda shape: pl.BlockSpec(shape, lambda j: (0,) * len(shape))
    return pl.pallas_call(
        _input_body,
        grid=(NBLK,),
        in_specs=[
            pl.BlockSpec((BLK, 2), lambda j: (j, 0)),
            pl.BlockSpec((BLK, 2), lambda j: (j, 0)),
            full((2, D)), full((1, D)), full((D, D)), full((1, D)), full((1, D)),
            full((2, D)), full((1, D)), full((D, D)), full((1, D)), full((1, D)),
        ],
        out_specs=pl.BlockSpec((BLK, D), lambda j: (j, 0)),
        out_shape=jax.ShapeDtypeStruct((NPAD, D), jnp.float32),
    )(ctrs_p, feats_p, w1c, b1c, w2c, gc, bc, w1s, b1s, w2s, gs, bs)


def _yall_body(feat_ref, w_ref, out_ref):
    out_ref[0] = jnp.dot(feat_ref[...], w_ref[0],
                         preferred_element_type=jnp.float32,
                         precision=lax.Precision.HIGHEST)


def _yall_stage(feat, wt6):
    return pl.pallas_call(
        _yall_body,
        grid=(N_REL, NBLK),
        in_specs=[
            pl.BlockSpec((BLK, D), lambda r, j: (j, 0)),
            pl.BlockSpec((1, D, D), lambda r, j: (r, 0, 0)),
        ],
        out_specs=pl.BlockSpec((1, BLK, D), lambda r, j: (r, j, 0)),
        out_shape=jax.ShapeDtypeStruct((N_REL, NPAD, D), jnp.float32),
    )(feat, wt6)


def _sc_edge_body(ytab, ucat, vcat, zeros, out,
                  acc, vidx_a, uidx_a, vidx_b, uidx_b, rows_a, rows_b,
                  sem_a, sem_b):
    c = lax.axis_index("c")
    s = lax.axis_index("s")
    r0 = s * ROWS_PER_TILE
    pltpu.sync_copy(zeros.at[pl.ds(r0, ROWS_PER_TILE)],
                    acc.at[pl.ds(r0, ROWS_PER_TILE)])
    plsc.subcore_barrier()

    my_cpt = lax.select(c == 0, CPTA, CPTB)
    ebase = pl.multiple_of(
        (lax.select(c == 0, s * CPTA, 16 * CPTA + s * CPTB)) * CHUNK, CHUNK)
    pltpu.sync_copy(vcat.at[pl.ds(ebase, CHUNK)], vidx_a)
    pltpu.sync_copy(ucat.at[pl.ds(ebase, CHUNK)], uidx_a)
    pltpu.async_copy(ytab.at[vidx_a], rows_a, sem_a)

    def chunk_step(k, vidx, uidx, rows, sem, vidx_n, uidx_n, rows_n, sem_n):
        @pl.when(k + 1 < my_cpt)
        def _():
            off = pl.multiple_of(ebase + (k + 1) * CHUNK, CHUNK)
            pltpu.sync_copy(vcat.at[pl.ds(off, CHUNK)], vidx_n)
            pltpu.sync_copy(ucat.at[pl.ds(off, CHUNK)], uidx_n)
            pltpu.async_copy(ytab.at[vidx_n], rows_n, sem_n)
        pltpu.make_async_copy(ytab.at[vidx], rows, sem).wait()
        pltpu.sync_copy(rows, acc.at[uidx], add=True)

    def outer(j, carry):
        chunk_step(2 * j, vidx_a, uidx_a, rows_a, sem_a,
                   vidx_b, uidx_b, rows_b, sem_b)
        chunk_step(2 * j + 1, vidx_b, uidx_b, rows_b, sem_b,
                   vidx_a, uidx_a, rows_a, sem_a)
        return carry

    lax.fori_loop(0, my_cpt // 2, outer, 0)

    plsc.subcore_barrier()
    pltpu.sync_copy(acc.at[pl.ds(r0, ROWS_PER_TILE)],
                    out.at[c, pl.ds(r0, ROWS_PER_TILE)])


@functools.cache
def _get_sc_kernel():
    return pl.kernel(
        _sc_edge_body,
        out_type=jax.ShapeDtypeStruct((2, NPAD, D), jnp.float32),
        mesh=plsc.VectorSubcoreMesh(core_axis_name="c", subcore_axis_name="s",
                                    num_cores=2, num_subcores=16),
        scratch_types=(
            [pltpu.VMEM_SHARED((NPAD, D), jnp.float32)]
            + [pltpu.VMEM((CHUNK,), jnp.int32)] * 4
            + [pltpu.VMEM((CHUNK, D), jnp.float32)] * 2
            + [pltpu.SemaphoreType.DMA] * 2
        ),
    )


def _sc_edge_stage(ytab, ucat, vcat, zeros):
    return _get_sc_kernel()(ytab, ucat, vcat, zeros)


def _norm_body(feat_ref, p_ref, wctr, g1, b1, wc2, g2, b2, out_ref):
    f = feat_ref[...]
    temp = jnp.dot(f, wctr[...], preferred_element_type=jnp.float32,
                   precision=lax.Precision.HIGHEST) + p_ref[0] + p_ref[1]
    t = jnp.maximum(_gn_block(temp, g1[...], b1[...]), 0.0)
    t = _gn_block(
        jnp.dot(t, wc2[...], preferred_element_type=jnp.float32,
                precision=lax.Precision.HIGHEST), g2[...], b2[...])
    out_ref[...] = jnp.maximum(t + f, 0.0)


def _norm_stage(feat, partials, wctr_t, g1, b1, wc2_t, g2, b2):
    full = lambda shape: pl.BlockSpec(shape, lambda j: (0,) * len(shape))
    return pl.pallas_call(
        _norm_body,
        grid=(NBLK,),
        in_specs=[
            pl.BlockSpec((BLK, D), lambda j: (j, 0)),
            pl.BlockSpec((2, BLK, D), lambda j: (0, j, 0)),
            full((D, D)), full((1, D)), full((1, D)),
            full((D, D)), full((1, D)), full((1, D)),
        ],
        out_specs=pl.BlockSpec((BLK, D), lambda j: (j, 0)),
        out_shape=jax.ShapeDtypeStruct((NPAD, D), jnp.float32),
    )(feat, partials, wctr_t, g1, b1, wc2_t, g2, b2)


def kernel(feats, ctrs, pre0_u, pre0_v, pre1_u, pre1_v, suc0_u, suc0_v,
           suc1_u, suc1_v, left_u, left_v, right_u, right_v, W_in1, b_in1,
           W_in2, g_in, be_in, W_seg1, b_seg1, W_seg2, g_seg, be_seg, W_ctr,
           W_pre, W_suc, W_left, W_right, g_norm, be_norm, W_ctr2, g_ctr2,
           be_ctr2):
    f32 = jnp.float32
    row = lambda x: x.reshape(1, D).astype(f32)

    ctrs_p = jnp.zeros((NPAD, 2), f32).at[:N_NODES].set(ctrs)
    feats_p = jnp.zeros((NPAD, 2), f32).at[:N_NODES].set(feats)

    feat = _input_stage(
        ctrs_p, feats_p,
        W_in1.T.astype(f32), row(b_in1), W_in2.T.astype(f32), row(g_in),
        row(be_in),
        W_seg1.T.astype(f32), row(b_seg1), W_seg2.T.astype(f32), row(g_seg),
        row(be_seg))

    dummy_u = (N_NODES
               + jnp.arange(E_PAD - E_TOTAL, dtype=jnp.int32) % NJUNK)
    uc = jnp.concatenate(
        [pre0_u.astype(jnp.int32), pre1_u.astype(jnp.int32),
         suc0_u.astype(jnp.int32), suc1_u.astype(jnp.int32),
         left_u.astype(jnp.int32), right_u.astype(jnp.int32), dummy_u])
    vc = jnp.concatenate([
        pre0_v, pre1_v + NPAD, suc0_v + 2 * NPAD, suc1_v + 3 * NPAD,
        left_v + 4 * NPAD, right_v + 5 * NPAD,
        jnp.zeros((E_PAD - E_TOTAL,), jnp.int32)]).astype(jnp.int32)

    wt6 = jnp.stack([W_pre[:, 0], W_pre[:, 1], W_suc[:, 0], W_suc[:, 1],
                     W_left, W_right], axis=1).swapaxes(-1, -2)
    wctr_t = W_ctr.swapaxes(-1, -2)
    wc2_t = W_ctr2.swapaxes(-1, -2)

    zeros = jnp.zeros((NPAD, D), f32)

    for i in range(4):
        yall = _yall_stage(feat, wt6[i])
        parts = _sc_edge_stage(yall.reshape(N_REL * NPAD, D), uc, vc, zeros)
        feat = _norm_stage(feat, parts, wctr_t[i],
                           row(g_norm[i]), row(be_norm[i]), wc2_t[i],
                           row(g_ctr2[i]), row(be_ctr2[i]))

    return feat[:N_NODES]


# speedup vs baseline: 1.1989x; 1.1989x over previous
import functools

import jax
import jax.numpy as jnp
from jax import lax
from jax.experimental import pallas as pl
from jax.experimental.pallas import tpu as pltpu
from jax.experimental.pallas import tpu_sc as plsc

N_NODES = 10000
D = 128
NPAD = 10240
N_REL = 6
E_TOTAL = 4 * 320000 + 2 * 40000
NW = 32
CHUNK = 128
GRP = 8
NBUF = 2
GLAG = 1 if NBUF < 4 else 2
CPTA = 404
CPTB = 264
CPT = (CPTA + CPTB) // 2
E_PAD = NW * CPT * CHUNK
NJUNK = NPAD - N_NODES
ROWS_PER_TILE = NPAD // 16
BLK = 512
NBLK = NPAD // BLK
EPS = 1e-5


def _gn_block(x, g, b):
    m = jnp.mean(x, axis=1, keepdims=True)
    v = jnp.mean((x - m) ** 2, axis=1, keepdims=True)
    return (x - m) * lax.rsqrt(v + EPS) * g + b


def _input_body(ctr_ref, ft_ref, w1c, b1c, w2c, gc, bc, w1s, b1s, w2s, gs, bs,
                out_ref):
    h = jnp.maximum(
        jnp.dot(ctr_ref[...], w1c[...], preferred_element_type=jnp.float32,
                precision=lax.Precision.HIGHEST) + b1c[...], 0.0)
    h = _gn_block(
        jnp.dot(h, w2c[...], preferred_element_type=jnp.float32,
                precision=lax.Precision.HIGHEST), gc[...], bc[...])
    s = jnp.maximum(
        jnp.dot(ft_ref[...], w1s[...], preferred_element_type=jnp.float32,
                precision=lax.Precision.HIGHEST) + b1s[...], 0.0)
    s = _gn_block(
        jnp.dot(s, w2s[...], preferred_element_type=jnp.float32,
                precision=lax.Precision.HIGHEST), gs[...], bs[...])
    out_ref[...] = jnp.maximum(h + s, 0.0)


def _input_stage(ctrs_p, feats_p, w1c, b1c, w2c, gc, bc, w1s, b1s, w2s, gs, bs):
    full = lambda shape: pl.BlockSpec(shape, lambda j: (0,) * len(shape))
    return pl.pallas_call(
        _input_body,
        grid=(NBLK,),
        in_specs=[
            pl.BlockSpec((BLK, 2), lambda j: (j, 0)),
            pl.BlockSpec((BLK, 2), lambda j: (j, 0)),
            full((2, D)), full((1, D)), full((D, D)), full((1, D)), full((1, D)),
            full((2, D)), full((1, D)), full((D, D)), full((1, D)), full((1, D)),
        ],
        out_specs=pl.BlockSpec((BLK, D), lambda j: (j, 0)),
        out_shape=jax.ShapeDtypeStruct((NPAD, D), jnp.float32),
    )(ctrs_p, feats_p, w1c, b1c, w2c, gc, bc, w1s, b1s, w2s, gs, bs)


def _yall_body(feat_ref, w_ref, out_ref):
    out_ref[0] = jnp.dot(feat_ref[...], w_ref[0],
                         preferred_element_type=jnp.float32,
                         precision=lax.Precision.HIGHEST)


def _yall_stage(feat, wt6):
    return pl.pallas_call(
        _yall_body,
        grid=(N_REL, NBLK),
        in_specs=[
            pl.BlockSpec((BLK, D), lambda r, j: (j, 0)),
            pl.BlockSpec((1, D, D), lambda r, j: (r, 0, 0)),
        ],
        out_specs=pl.BlockSpec((1, BLK, D), lambda r, j: (r, j, 0)),
        out_shape=jax.ShapeDtypeStruct((N_REL, NPAD, D), jnp.float32),
    )(feat, wt6)


def _sc_edge_body(ytab, ucat, vcat, zeros, out,
                  acc, vidx_a, uidx_a, vidx_b, uidx_b, rows_a, rows_b,
                  sem_a, sem_b):
    c = lax.axis_index("c")
    s = lax.axis_index("s")
    r0 = s * ROWS_PER_TILE
    pltpu.sync_copy(zeros.at[pl.ds(r0, ROWS_PER_TILE)],
                    acc.at[pl.ds(r0, ROWS_PER_TILE)])
    plsc.subcore_barrier()

    my_cpt = lax.select(c == 0, CPTA, CPTB)
    ebase = pl.multiple_of(
        (lax.select(c == 0, s * CPTA, 16 * CPTA + s * CPTB)) * CHUNK, CHUNK)
    pltpu.sync_copy(vcat.at[pl.ds(ebase, CHUNK)], vidx_a)
    pltpu.sync_copy(ucat.at[pl.ds(ebase, CHUNK)], uidx_a)
    pltpu.async_copy(ytab.at[vidx_a], rows_a, sem_a)

    def chunk_step(k, vidx, uidx, rows, sem, vidx_n, uidx_n, rows_n, sem_n):
        @pl.when(k + 1 < my_cpt)
        def _():
            off = pl.multiple_of(ebase + (k + 1) * CHUNK, CHUNK)
            pltpu.sync_copy(vcat.at[pl.ds(off, CHUNK)], vidx_n)
            pltpu.sync_copy(ucat.at[pl.ds(off, CHUNK)], uidx_n)
            pltpu.async_copy(ytab.at[vidx_n], rows_n, sem_n)
        pltpu.make_async_copy(ytab.at[vidx], rows, sem).wait()
        pltpu.sync_copy(rows, acc.at[uidx], add=True)

    def outer(j, carry):
        chunk_step(2 * j, vidx_a, uidx_a, rows_a, sem_a,
                   vidx_b, uidx_b, rows_b, sem_b)
        chunk_step(2 * j + 1, vidx_b, uidx_b, rows_b, sem_b,
                   vidx_a, uidx_a, rows_a, sem_a)
        return carry

    lax.fori_loop(0, my_cpt // 2, outer, 0)

    plsc.subcore_barrier()
    pltpu.sync_copy(acc.at[pl.ds(r0, ROWS_PER_TILE)],
                    out.at[c, pl.ds(r0, ROWS_PER_TILE)])


@functools.cache
def _get_sc_kernel():
    return pl.kernel(
        _sc_edge_body,
        out_type=jax.ShapeDtypeStruct((2, NPAD, D), jnp.float32),
        mesh=plsc.VectorSubcoreMesh(core_axis_name="c", subcore_axis_name="s",
                                    num_cores=2, num_subcores=16),
        scratch_types=(
            [pltpu.VMEM_SHARED((NPAD, D), jnp.float32)]
            + [pltpu.VMEM((CHUNK,), jnp.int32)] * 4
            + [pltpu.VMEM((CHUNK, D), jnp.float32)] * 2
            + [pltpu.SemaphoreType.DMA] * 2
        ),
    )


def _sc_edge_stage(ytab, ucat, vcat, zeros):
    return _get_sc_kernel()(ytab, ucat, vcat, zeros)


def _norm_body(feat_ref, p_ref, wctr, g1, b1, wc2, g2, b2, out_ref):
    f = feat_ref[...]
    temp = jnp.dot(f, wctr[...], preferred_element_type=jnp.float32,
                   precision=lax.Precision.HIGHEST) + p_ref[0] + p_ref[1]
    t = jnp.maximum(_gn_block(temp, g1[...], b1[...]), 0.0)
    t = _gn_block(
        jnp.dot(t, wc2[...], preferred_element_type=jnp.float32,
                precision=lax.Precision.HIGHEST), g2[...], b2[...])
    out_ref[...] = jnp.maximum(t + f, 0.0)


def _norm_stage(feat, partials, wctr_t, g1, b1, wc2_t, g2, b2):
    full = lambda shape: pl.BlockSpec(shape, lambda j: (0,) * len(shape))
    return pl.pallas_call(
        _norm_body,
        grid=(NBLK,),
        in_specs=[
            pl.BlockSpec((BLK, D), lambda j: (j, 0)),
            pl.BlockSpec((2, BLK, D), lambda j: (0, j, 0)),
            full((D, D)), full((1, D)), full((1, D)),
            full((D, D)), full((1, D)), full((1, D)),
        ],
        out_specs=pl.BlockSpec((BLK, D), lambda j: (j, 0)),
        out_shape=jax.ShapeDtypeStruct((NPAD, D), jnp.float32),
    )(feat, partials, wctr_t, g1, b1, wc2_t, g2, b2)


def kernel(feats, ctrs, pre0_u, pre0_v, pre1_u, pre1_v, suc0_u, suc0_v,
           suc1_u, suc1_v, left_u, left_v, right_u, right_v, W_in1, b_in1,
           W_in2, g_in, be_in, W_seg1, b_seg1, W_seg2, g_seg, be_seg, W_ctr,
           W_pre, W_suc, W_left, W_right, g_norm, be_norm, W_ctr2, g_ctr2,
           be_ctr2):
    f32 = jnp.float32
    row = lambda x: x.reshape(1, D).astype(f32)

    ctrs_p = jnp.zeros((NPAD, 2), f32).at[:N_NODES].set(ctrs)
    feats_p = jnp.zeros((NPAD, 2), f32).at[:N_NODES].set(feats)

    feat = _input_stage(
        ctrs_p, feats_p,
        W_in1.T.astype(f32), row(b_in1), W_in2.T.astype(f32), row(g_in),
        row(be_in),
        W_seg1.T.astype(f32), row(b_seg1), W_seg2.T.astype(f32), row(g_seg),
        row(be_seg))

    dummy_u = (N_NODES
               + jnp.arange(E_PAD - E_TOTAL, dtype=jnp.int32) % NJUNK)
    uc = jnp.concatenate(
        [pre0_u.astype(jnp.int32), pre1_u.astype(jnp.int32),
         suc0_u.astype(jnp.int32), suc1_u.astype(jnp.int32),
         left_u.astype(jnp.int32), right_u.astype(jnp.int32), dummy_u])
    vc = jnp.concatenate([
        pre0_v, pre1_v + NPAD, suc0_v + 2 * NPAD, suc1_v + 3 * NPAD,
        left_v + 4 * NPAD, right_v + 5 * NPAD,
        jnp.zeros((E_PAD - E_TOTAL,), jnp.int32)]).astype(jnp.int32)

    wt6 = jnp.stack([W_pre[:, 0], W_pre[:, 1], W_suc[:, 0], W_suc[:, 1],
                     W_left, W_right], axis=1).swapaxes(-1, -2)
    wctr_t = W_ctr.swapaxes(-1, -2)
    wc2_t = W_ctr2.swapaxes(-1, -2)

    zeros = jnp.zeros((NPAD, D), f32)

    for i in range(4):
        yall = _yall_stage(feat, wt6[i])
        parts = _sc_edge_stage(yall.reshape(N_REL * NPAD, D), uc, vc, zeros)
        feat = _norm_stage(feat, parts, wctr_t[i],
                           row(g_norm[i]), row(be_norm[i]), wc2_t[i],
                           row(g_ctr2[i]), row(be_ctr2[i]))

    return feat[:N_NODES]
